# baseline (speedup 1.0000x reference)
import functools

import jax
import jax.numpy as jnp
from jax import lax
from jax.experimental import pallas as pl
from jax.experimental.pallas import tpu as pltpu
from jax.experimental.pallas import tpu_sc as plsc

N = 10000
E = 160000
DIN = 128
DOUT = 256
CC = DIN + DOUT
NET = 2
STEPS = 6
NG = 50
LG = N // NG

NHALF = N // 2
SH = 5120
DUMMY = NHALF
K = 128
NSUB = 16
EPT = ((E // NSUB + K - 1) // K) * K
EP = EPT * NSUB
NITER = EPT // K

_f32 = jnp.float32


def _dot(x, w):
    return lax.dot_general(x, w, (((1,), (1,)), ((), ())),
                           preferred_element_type=_f32)


def _prep_body(src_ref, et_ref, gidx_ref):
    gidx_ref[...] = et_ref[...] * N + src_ref[...]


def _prep(src, et):
    rows = EP // 128
    shp = (rows, 128)
    return pl.pallas_call(
        _prep_body,
        grid=(1,),
        in_specs=[pl.BlockSpec(shp, lambda i: (0, 0))] * 2,
        out_specs=pl.BlockSpec(shp, lambda i: (0, 0)),
        out_shape=jax.ShapeDtypeStruct(shp, jnp.int32),
    )(src.reshape(shp), et.reshape(shp))


BLK = 1000


def _mnode(h, We_ref, be_ref, table_ref):
    for t in range(NET):
        table_ref[t] = _dot(h, We_ref[t]) + be_ref[t][None, :]


def _init_body(f_ref, We_ref, be_ref, h_ref, table_ref):
    f = f_ref[...]
    h = jnp.concatenate([f, jnp.zeros((BLK, DOUT - DIN), _f32)], axis=1)
    h_ref[...] = h
    _mnode(h, We_ref, be_ref, table_ref)


def _gru_core(a_ref, h_ref, Wih_ref, Whh_ref, bih_ref, bhh_ref):
    a = a_ref[...]
    h = h_ref[...]
    gi = _dot(a, Wih_ref[...]) + bih_ref[...]
    gh = _dot(h, Whh_ref[...]) + bhh_ref[...]
    r = jax.nn.sigmoid(gi[:, :DOUT] + gh[:, :DOUT])
    z = jax.nn.sigmoid(gi[:, DOUT:2 * DOUT] + gh[:, DOUT:2 * DOUT])
    n = jnp.tanh(gi[:, 2 * DOUT:] + r * gh[:, 2 * DOUT:])
    return (1.0 - z) * n + z * h


def _gru_mnode_body(a_ref, h_ref, Wih_ref, Whh_ref, bih_ref, bhh_ref,
                    We_ref, be_ref, h2_ref, table_ref):
    h2 = _gru_core(a_ref, h_ref, Wih_ref, Whh_ref, bih_ref, bhh_ref)
    h2_ref[...] = h2
    _mnode(h2, We_ref, be_ref, table_ref)


def _gru_body(a_ref, h_ref, Wih_ref, Whh_ref, bih_ref, bhh_ref, h2_ref):
    h2_ref[...] = _gru_core(a_ref, h_ref, Wih_ref, Whh_ref, bih_ref, bhh_ref)


def _full(shape):
    nd = len(shape)
    return pl.BlockSpec(shape, lambda i, _n=nd: (0,) * _n)


def _row_spec(cols):
    return pl.BlockSpec((BLK, cols), lambda i: (i, 0))


def _init_call(features, W_e, b_e):
    return pl.pallas_call(
        _init_body,
        grid=(N // BLK,),
        in_specs=[_row_spec(DIN), _full((NET, DOUT, DOUT)), _full((NET, DOUT))],
        out_specs=[_row_spec(DOUT),
                   pl.BlockSpec((NET, BLK, DOUT), lambda i: (0, i, 0))],
        out_shape=[jax.ShapeDtypeStruct((N, DOUT), _f32),
                   jax.ShapeDtypeStruct((NET, N, DOUT), _f32)],
    )(features, W_e, b_e)


def _gru_mnode_call(a, h, W_ih, W_hh, b_ih2, b_hh2, W_e, b_e):
    return pl.pallas_call(
        _gru_mnode_body,
        grid=(N // BLK,),
        in_specs=[_row_spec(DOUT), _row_spec(DOUT),
                  _full((3 * DOUT, DOUT)), _full((3 * DOUT, DOUT)),
                  _full((1, 3 * DOUT)), _full((1, 3 * DOUT)),
                  _full((NET, DOUT, DOUT)), _full((NET, DOUT))],
        out_specs=[_row_spec(DOUT),
                   pl.BlockSpec((NET, BLK, DOUT), lambda i: (0, i, 0))],
        out_shape=[jax.ShapeDtypeStruct((N, DOUT), _f32),
                   jax.ShapeDtypeStruct((NET, N, DOUT), _f32)],
    )(a, h, W_ih, W_hh, b_ih2, b_hh2, W_e, b_e)


def _gru_call(a, h, W_ih, W_hh, b_ih2, b_hh2):
    return pl.pallas_call(
        _gru_body,
        grid=(N // BLK,),
        in_specs=[_row_spec(DOUT), _row_spec(DOUT),
                  _full((3 * DOUT, DOUT)), _full((3 * DOUT, DOUT)),
                  _full((1, 3 * DOUT)), _full((1, 3 * DOUT))],
        out_specs=_row_spec(DOUT),
        out_shape=jax.ShapeDtypeStruct((N, DOUT), _f32),
    )(a, h, W_ih, W_hh, b_ih2, b_hh2)


NW = 32
RPT = 313
CAP = 8192
CH = 64
SB = 1024
PADV = 1 << 20
NOUT = NW * RPT


def _mesh():
    return plsc.VectorSubcoreMesh(core_axis_name="c", subcore_axis_name="s")


_GDN = lax.GatherDimensionNumbers(offset_dims=(), collapsed_slice_dims=(0,),
                                  start_index_map=(0,))


def _lane_gather(x, idx):
    return lax.gather(x, idx[:, None], _GDN, (1,),
                      mode=lax.GatherScatterMode.PROMISE_IN_BOUNDS)


def _scprep_body(gidx_hbm, dst_hbm, cpk_hbm, cnt_hbm,
                 gst_v, dst_v, cpk_v, cw_v, sg0, sg1, sd0, sd1):
    c = lax.axis_index("c")
    s = lax.axis_index("s")
    w = c * 16 + s
    lo = w * RPT
    lanes = lax.iota(jnp.int32, 16)
    lane15 = jnp.full((16,), 15, jnp.int32)
    sg = [sg0, sg1]
    sd = [sd0, sd1]

    NB = EP // SB

    def start(blk, b):
        pltpu.async_copy(gidx_hbm.at[pl.ds(blk * SB, SB)],
                         gst_v.at[b], sg[b])
        pltpu.async_copy(dst_hbm.at[pl.ds(blk * SB, SB)],
                         dst_v.at[b], sd[b])

    def wait(blk, b):
        pltpu.make_async_copy(gidx_hbm.at[pl.ds(blk * SB, SB)],
                              gst_v.at[b], sg[b]).wait()
        pltpu.make_async_copy(dst_hbm.at[pl.ds(blk * SB, SB)],
                              dst_v.at[b], sd[b]).wait()

    start(0, 0)

    def pair(g, off):
        for b in range(2):
            blk = g * 2 + b
            wait(blk, b)

            @pl.when(blk + 1 < NB)
            def _():
                start(blk + 1, 1 - b)

            for k in range(SB // 16):
                d16 = dst_v[b, pl.ds(k * 16, 16)]
                g16 = gst_v[b, pl.ds(k * 16, 16)]
                dl = d16 - lo
                m = (dl >= 0) & (dl < RPT)
                pk = g16 * 512 + dl
                x = jnp.where(m, 1, 0).astype(jnp.int32)
                for sh in (1, 2, 4, 8):
                    gv = _lane_gather(x, jnp.maximum(lanes - sh, 0))
                    x = x + jnp.where(lanes >= sh, gv, 0)
                plsc.store_scatter(cpk_v, [off + x - 1], pk, mask=m)
                off = off + _lane_gather(x, lane15)
        return off

    cnt = lax.fori_loop(0, NB // 2, pair, jnp.zeros((16,), jnp.int32))

    dummy = jnp.full((16,), RPT, jnp.int32)
    ones = jnp.full((16,), True)
    for t in range(CH // 16):
        plsc.store_scatter(cpk_v, [cnt + t * 16 + lanes], dummy, mask=ones)

    pltpu.sync_copy(cpk_v, cpk_hbm.at[w])
    cw_v[...] = cnt
    pltpu.sync_copy(cw_v, cnt_hbm.at[w])


def _sc_prep(gidx, dstp):
    fn = pl.kernel(
        _scprep_body,
        mesh=_mesh(),
        compiler_params=pltpu.CompilerParams(needs_layout_passes=False),
        out_type=[jax.ShapeDtypeStruct((NW, CAP), jnp.int32),
                  jax.ShapeDtypeStruct((NW, 16), jnp.int32)],
        scratch_types=[
            pltpu.VMEM((2, SB), jnp.int32),
            pltpu.VMEM((2, SB), jnp.int32),
            pltpu.VMEM((CAP,), jnp.int32),
            pltpu.VMEM((16,), jnp.int32),
            pltpu.SemaphoreType.DMA,
            pltpu.SemaphoreType.DMA,
            pltpu.SemaphoreType.DMA,
            pltpu.SemaphoreType.DMA,
        ],
    )
    return fn(gidx, dstp)


def _scstep_body(cpk_hbm, cnt_hbm, table_hbm, out_hbm,
                 pk_v, gl_v, rows_v, cw_v, acc_v, s0, s1):
    c = lax.axis_index("c")
    s = lax.axis_index("s")
    w = c * 16 + s
    sems = [s0, s1]

    def zr(i, carry):
        acc_v[pl.ds(i * 16, 16)] = jnp.zeros((16,), _f32)
        return carry

    lax.fori_loop(0, (RPT + 1) * DOUT // 16, zr, 0)

    pltpu.sync_copy(cnt_hbm.at[w], cw_v)
    cnt = cw_v[...][0]
    nch = (cnt + CH - 1) // CH

    pltpu.sync_copy(cpk_hbm.at[w], pk_v)

    def unp(i, carry):
        gl_v[pl.ds(i * 16, 16)] = pk_v[pl.ds(i * 16, 16)] >> 9
        return carry

    lax.fori_loop(0, CAP // 16, unp, 0)

    def start(ci, b):
        pltpu.async_copy(table_hbm.at[gl_v.at[pl.ds(ci * CH, CH)]],
                         rows_v.at[b], sems[b])

    def wait(ci, b):
        pltpu.make_async_copy(table_hbm.at[gl_v.at[pl.ds(ci * CH, CH)]],
                              rows_v.at[b], sems[b]).wait()

    @pl.when(nch > 0)
    def _():
        start(0, 0)

    def pair(g, carry):
        for b in range(2):
            ci = g * 2 + b

            @pl.when(ci < nch)
            def _():
                wait(ci, b)

                @pl.when(ci + 1 < nch)
                def _():
                    start(ci + 1, 1 - b)

                def grp(gi, carry):
                    pk16 = pk_v[pl.ds(ci * CH + gi * 16, 16)]
                    d16 = (pk16 & 511) * DOUT
                    for l in range(16):
                        doff = d16[l]
                        xs = [rows_v[b, gi * 16 + l, pl.ds(j * 16, 16)]
                              for j in range(DOUT // 16)]
                        for j in range(DOUT // 16):
                            plsc.addupdate(
                                acc_v.at[pl.ds(doff + j * 16, 16)], xs[j])
                    return carry

                lax.fori_loop(0, CH // 16, grp, 0)
        return carry

    lax.fori_loop(0, (nch + 1) // 2, pair, 0)

    pltpu.sync_copy(acc_v.at[pl.ds(0, RPT * DOUT)],
                    out_hbm.at[pl.ds(w * RPT * DOUT, RPT * DOUT)])


def _sc_step(cpk, cnts, table_flat):
    fn = pl.kernel(
        _scstep_body,
        mesh=_mesh(),
        compiler_params=pltpu.CompilerParams(needs_layout_passes=False),
        out_type=jax.ShapeDtypeStruct((NOUT * DOUT,), _f32),
        scratch_types=[
            pltpu.VMEM((CAP,), jnp.int32),
            pltpu.VMEM((CAP,), jnp.int32),
            pltpu.VMEM((2, CH, DOUT), _f32),
            pltpu.VMEM((16,), jnp.int32),
            pltpu.VMEM(((RPT + 1) * DOUT,), _f32),
            pltpu.SemaphoreType.DMA,
            pltpu.SemaphoreType.DMA,
        ],
    )
    return fn(cpk, cnts, table_flat)


def _message_pass(cpk, cnts, table_flat):
    return _sc_step(cpk, cnts, table_flat).reshape(NOUT, DOUT)[:N]


L1 = LG - 2
L1P = (L1 - 3) // 2 + 1
L2P = (L1P - 2) // 2 + 1
CNT1 = NG * L1
CNT2 = NG * L1P
EPS = 1e-5


def _conv3(x, w_ref, b_ref, width):
    acc = b_ref[...] * jnp.ones((L1, width), _f32)
    for d in range(3):
        acc = acc + _dot(x[d:d + L1], w_ref[d])
    return acc


def _stats_acc(g, y, sum_ref, sq_ref):
    @pl.when(g == 0)
    def _():
        sum_ref[...] = jnp.zeros_like(sum_ref)
        sq_ref[...] = jnp.zeros_like(sq_ref)
    sum_ref[...] += jnp.sum(y, axis=0, keepdims=True)
    sq_ref[...] += jnp.sum(y * y, axis=0, keepdims=True)


def _r1_body(h_ref, x_ref, w1_ref, b1_ref, wc1_ref, bc1_ref,
             y1_ref, z1_ref, ys_ref, yq_ref, zs_ref, zq_ref):
    g = pl.program_id(0)
    hb = h_ref[0]
    xb = x_ref[0]
    y = _conv3(hb, w1_ref, b1_ref, DOUT)
    cb = jnp.concatenate([hb, xb], axis=1)
    z = _conv3(cb, wc1_ref, bc1_ref, CC)
    y1_ref[0] = y
    z1_ref[0] = z
    _stats_acc(g, y, ys_ref, yq_ref)
    _stats_acc(g, z, zs_ref, zq_ref)


def _bn_relu(x, s_ref, q_ref, g_ref, b_ref, cnt):
    m = s_ref[...] / cnt
    v = q_ref[...] / cnt - m * m
    scale = g_ref[...] * lax.rsqrt(v + EPS)
    return jnp.maximum((x - m) * scale + b_ref[...], 0.0)


def _pool32(x, lout):
    m1 = jnp.max(x[:2 * lout].reshape(lout, 2, -1), axis=1)
    m2 = x[2:2 + 2 * lout].reshape(lout, 2, -1)[:, 0]
    return jnp.maximum(m1, m2)


def _pool22(x, lout):
    return jnp.max(x[:2 * lout].reshape(lout, 2, -1), axis=1)


def _r2_body(y1_ref, z1_ref, ys_ref, yq_ref, zs_ref, zq_ref,
             bng_ref, bnb_ref, bncg_ref, bncb_ref, w2_ref, wc2_ref,
             b2_ref, bc2_ref,
             y2_ref, z2_ref, ys2_ref, yq2_ref, zs2_ref, zq2_ref):
    g = pl.program_id(0)
    yn = _bn_relu(y1_ref[0], ys_ref, yq_ref, bng_ref, bnb_ref, float(CNT1))
    zn = _bn_relu(z1_ref[0], zs_ref, zq_ref, bncg_ref, bncb_ref, float(CNT1))
    yp = _pool32(yn, L1P)
    zp = _pool32(zn, L1P)
    y2 = _dot(yp, w2_ref[...]) + b2_ref[...]
    z2 = _dot(zp, wc2_ref[...]) + bc2_ref[...]
    y2_ref[0] = y2
    z2_ref[0] = z2
    _stats_acc(g, y2, ys2_ref, yq2_ref)
    _stats_acc(g, z2, zs2_ref, zq2_ref)


def _r3_body(y2_ref, z2_ref, ys2_ref, yq2_ref, zs2_ref, zq2_ref,
             bng_ref, bnb_ref, bncg_ref, bncb_ref,
             mly_ref, mlyb_ref, mlz_ref, mlzb_ref, out_ref):
    yn = _bn_relu(y2_ref[0], ys2_ref, yq2_ref, bng_ref, bnb_ref, float(CNT2))
    zn = _bn_relu(z2_ref[0], zs2_ref, zq2_ref, bncg_ref, bncb_ref, float(CNT2))
    yp = _pool22(yn, L2P)
    zp = _pool22(zn, L2P)
    yv = jnp.sum(yp * mly_ref[0][None, :], axis=1) + mlyb_ref[0, 0]
    zv = jnp.sum(zp * mlz_ref[0][None, :], axis=1) + mlzb_ref[0, 0]
    avg = jnp.mean(yv * zv)
    out_ref[...] = jnp.full((1, 8, 128), jax.nn.sigmoid(avg), _f32)


def _g_spec(l, cols):
    return pl.BlockSpec((1, l, cols), lambda g: (g, 0, 0))


def _readout(h, features, conv1_w, conv1_b, conv2_w, conv2_b,
             convc1_w, convc1_b, convc2_w, convc2_b,
             bn_g, bn_b, bnc_g, bnc_b, mly_w, mly_b, mlz_w, mlz_b):
    h_i = h.reshape(NG, LG, DOUT)
    x_i = features.reshape(NG, LG, DIN)
    w1t = jnp.transpose(conv1_w, (2, 0, 1))
    wc1t = jnp.transpose(convc1_w, (2, 0, 1))
    w2 = conv2_w[:, :, 0]
    wc2 = convc2_w[:, :, 0]

    y1, z1, ys, yq, zs, zq = pl.pallas_call(
        _r1_body,
        grid=(NG,),
        in_specs=[_g_spec(LG, DOUT), _g_spec(LG, DIN),
                  _full((3, DOUT, DOUT)), _full((1, DOUT)),
                  _full((3, CC, CC)), _full((1, CC))],
        out_specs=[_g_spec(L1, DOUT), _g_spec(L1, CC),
                   _full((1, DOUT)), _full((1, DOUT)),
                   _full((1, CC)), _full((1, CC))],
        out_shape=[jax.ShapeDtypeStruct((NG, L1, DOUT), _f32),
                   jax.ShapeDtypeStruct((NG, L1, CC), _f32),
                   jax.ShapeDtypeStruct((1, DOUT), _f32),
                   jax.ShapeDtypeStruct((1, DOUT), _f32),
                   jax.ShapeDtypeStruct((1, CC), _f32),
                   jax.ShapeDtypeStruct((1, CC), _f32)],
    )(h_i, x_i, w1t, conv1_b.reshape(1, DOUT), wc1t, convc1_b.reshape(1, CC))

    y2, z2, ys2, yq2, zs2, zq2 = pl.pallas_call(
        _r2_body,
        grid=(NG,),
        in_specs=[_g_spec(L1, DOUT), _g_spec(L1, CC),
                  _full((1, DOUT)), _full((1, DOUT)),
                  _full((1, CC)), _full((1, CC)),
                  _full((1, DOUT)), _full((1, DOUT)),
                  _full((1, CC)), _full((1, CC)),
                  _full((DOUT, DOUT)), _full((CC, CC)),
                  _full((1, DOUT)), _full((1, CC))],
        out_specs=[_g_spec(L1P, DOUT), _g_spec(L1P, CC),
                   _full((1, DOUT)), _full((1, DOUT)),
                   _full((1, CC)), _full((1, CC))],
        out_shape=[jax.ShapeDtypeStruct((NG, L1P, DOUT), _f32),
                   jax.ShapeDtypeStruct((NG, L1P, CC), _f32),
                   jax.ShapeDtypeStruct((1, DOUT), _f32),
                   jax.ShapeDtypeStruct((1, DOUT), _f32),
                   jax.ShapeDtypeStruct((1, CC), _f32),
                   jax.ShapeDtypeStruct((1, CC), _f32)],
    )(y1, z1, ys, yq, zs, zq,
      bn_g.reshape(1, DOUT), bn_b.reshape(1, DOUT),
      bnc_g.reshape(1, CC), bnc_b.reshape(1, CC),
      w2, wc2, conv2_b.reshape(1, DOUT), convc2_b.reshape(1, CC))

    out = pl.pallas_call(
        _r3_body,
        grid=(NG,),
        in_specs=[_g_spec(L1P, DOUT), _g_spec(L1P, CC),
                  _full((1, DOUT)), _full((1, DOUT)),
                  _full((1, CC)), _full((1, CC)),
                  _full((1, DOUT)), _full((1, DOUT)),
                  _full((1, CC)), _full((1, CC)),
                  _full((1, DOUT)), _full((1, 128)),
                  _full((1, CC)), _full((1, 128))],
        out_specs=pl.BlockSpec((1, 8, 128), lambda g: (g, 0, 0)),
        out_shape=jax.ShapeDtypeStruct((NG, 8, 128), _f32),
    )(y2, z2, ys2, yq2, zs2, zq2,
      bn_g.reshape(1, DOUT), bn_b.reshape(1, DOUT),
      bnc_g.reshape(1, CC), bnc_b.reshape(1, CC),
      mly_w, jnp.broadcast_to(mly_b.reshape(1, 1), (1, 128)),
      mlz_w, jnp.broadcast_to(mlz_b.reshape(1, 1), (1, 128)))

    return out[:, 0, 0]


def kernel(features, edge_index, edge_types, num_graphs, W_e, b_e, W_ih,
           W_hh, b_ih, b_hh, conv1_w, conv1_b, conv2_w, conv2_b, convc1_w,
           convc1_b, convc2_w, convc2_b, bn_g, bn_b, bnc_g, bnc_b, mly_w,
           mly_b, mlz_w, mlz_b):
    src = jnp.pad(edge_index[0].astype(jnp.int32), (0, EP - E))
    dstp = jnp.pad(edge_index[1].astype(jnp.int32), (0, EP - E),
                   constant_values=PADV)
    et = jnp.pad(edge_types.astype(jnp.int32), (0, EP - E))

    gidx = _prep(src, et).reshape(EP)
    cpk, cnts = _sc_prep(gidx, dstp)

    b_ih2 = b_ih.reshape(1, 3 * DOUT)
    b_hh2 = b_hh.reshape(1, 3 * DOUT)

    h, table = _init_call(features, W_e, b_e)
    for step in range(STEPS):
        a = _message_pass(cpk, cnts, table.reshape(NET * N, DOUT))
        if step < STEPS - 1:
            h, table = _gru_mnode_call(a, h, W_ih, W_hh, b_ih2, b_hh2,
                                       W_e, b_e)
        else:
            h = _gru_call(a, h, W_ih, W_hh, b_ih2, b_hh2)

    return _readout(h, features, conv1_w, conv1_b, conv2_w, conv2_b,
                    convc1_w, convc1_b, convc2_w, convc2_b,
                    bn_g, bn_b, bnc_g, bnc_b, mly_w, mly_b, mlz_w, mlz_b)


# speedup vs baseline: 1.1743x; 1.1743x over previous
import functools

import jax
import jax.numpy as jnp
from jax import lax
from jax.experimental import pallas as pl
from jax.experimental.pallas import tpu as pltpu
from jax.experimental.pallas import tpu_sc as plsc

N = 10000
E = 160000
DIN = 128
DOUT = 256
CC = DIN + DOUT
NET = 2
STEPS = 6
NG = 50
LG = N // NG

NHALF = N // 2
SH = 5120
DUMMY = NHALF
K = 128
NSUB = 16
EPT = ((E // NSUB + K - 1) // K) * K
EP = EPT * NSUB
NITER = EPT // K

_f32 = jnp.float32


def _dot(x, w):
    return lax.dot_general(x, w, (((1,), (1,)), ((), ())),
                           preferred_element_type=_f32)


def _prep_body(src_ref, et_ref, gidx_ref):
    gidx_ref[...] = et_ref[...] * N + src_ref[...]


def _prep(src, et):
    rows = EP // 128
    shp = (rows, 128)
    return pl.pallas_call(
        _prep_body,
        grid=(1,),
        in_specs=[pl.BlockSpec(shp, lambda i: (0, 0))] * 2,
        out_specs=pl.BlockSpec(shp, lambda i: (0, 0)),
        out_shape=jax.ShapeDtypeStruct(shp, jnp.int32),
    )(src.reshape(shp), et.reshape(shp))


BLK = 1000


def _mnode(h, We_ref, be_ref, table_ref):
    for t in range(NET):
        table_ref[t] = _dot(h, We_ref[t]) + be_ref[t][None, :]


def _init_body(f_ref, We_ref, be_ref, h_ref, table_ref):
    f = f_ref[...]
    h = jnp.concatenate([f, jnp.zeros((BLK, DOUT - DIN), _f32)], axis=1)
    h_ref[...] = h
    _mnode(h, We_ref, be_ref, table_ref)


def _gru_core(a_ref, h_ref, Wih_ref, Whh_ref, bih_ref, bhh_ref):
    a = a_ref[...]
    h = h_ref[...]
    gi = _dot(a, Wih_ref[...]) + bih_ref[...]
    gh = _dot(h, Whh_ref[...]) + bhh_ref[...]
    r = jax.nn.sigmoid(gi[:, :DOUT] + gh[:, :DOUT])
    z = jax.nn.sigmoid(gi[:, DOUT:2 * DOUT] + gh[:, DOUT:2 * DOUT])
    n = jnp.tanh(gi[:, 2 * DOUT:] + r * gh[:, 2 * DOUT:])
    return (1.0 - z) * n + z * h


def _gru_mnode_body(a_ref, h_ref, Wih_ref, Whh_ref, bih_ref, bhh_ref,
                    We_ref, be_ref, h2_ref, table_ref):
    h2 = _gru_core(a_ref, h_ref, Wih_ref, Whh_ref, bih_ref, bhh_ref)
    h2_ref[...] = h2
    _mnode(h2, We_ref, be_ref, table_ref)


def _gru_body(a_ref, h_ref, Wih_ref, Whh_ref, bih_ref, bhh_ref, h2_ref):
    h2_ref[...] = _gru_core(a_ref, h_ref, Wih_ref, Whh_ref, bih_ref, bhh_ref)


def _full(shape):
    nd = len(shape)
    return pl.BlockSpec(shape, lambda i, _n=nd: (0,) * _n)


def _row_spec(cols):
    return pl.BlockSpec((BLK, cols), lambda i: (i, 0))


def _init_call(features, W_e, b_e):
    return pl.pallas_call(
        _init_body,
        grid=(N // BLK,),
        in_specs=[_row_spec(DIN), _full((NET, DOUT, DOUT)), _full((NET, DOUT))],
        out_specs=[_row_spec(DOUT),
                   pl.BlockSpec((NET, BLK, DOUT), lambda i: (0, i, 0))],
        out_shape=[jax.ShapeDtypeStruct((N, DOUT), _f32),
                   jax.ShapeDtypeStruct((NET, N, DOUT), _f32)],
    )(features, W_e, b_e)


def _gru_mnode_call(a, h, W_ih, W_hh, b_ih2, b_hh2, W_e, b_e):
    return pl.pallas_call(
        _gru_mnode_body,
        grid=(N // BLK,),
        in_specs=[_row_spec(DOUT), _row_spec(DOUT),
                  _full((3 * DOUT, DOUT)), _full((3 * DOUT, DOUT)),
                  _full((1, 3 * DOUT)), _full((1, 3 * DOUT)),
                  _full((NET, DOUT, DOUT)), _full((NET, DOUT))],
        out_specs=[_row_spec(DOUT),
                   pl.BlockSpec((NET, BLK, DOUT), lambda i: (0, i, 0))],
        out_shape=[jax.ShapeDtypeStruct((N, DOUT), _f32),
                   jax.ShapeDtypeStruct((NET, N, DOUT), _f32)],
    )(a, h, W_ih, W_hh, b_ih2, b_hh2, W_e, b_e)


def _gru_call(a, h, W_ih, W_hh, b_ih2, b_hh2):
    return pl.pallas_call(
        _gru_body,
        grid=(N // BLK,),
        in_specs=[_row_spec(DOUT), _row_spec(DOUT),
                  _full((3 * DOUT, DOUT)), _full((3 * DOUT, DOUT)),
                  _full((1, 3 * DOUT)), _full((1, 3 * DOUT))],
        out_specs=_row_spec(DOUT),
        out_shape=jax.ShapeDtypeStruct((N, DOUT), _f32),
    )(a, h, W_ih, W_hh, b_ih2, b_hh2)


NW = 32
RPT = 313
CAP = 8192
CH = 64
SB = 1024
PADV = 1 << 20
NOUT = NW * RPT


def _mesh():
    return plsc.VectorSubcoreMesh(core_axis_name="c", subcore_axis_name="s")


_GDN = lax.GatherDimensionNumbers(offset_dims=(), collapsed_slice_dims=(0,),
                                  start_index_map=(0,))


def _lane_gather(x, idx):
    return lax.gather(x, idx[:, None], _GDN, (1,),
                      mode=lax.GatherScatterMode.PROMISE_IN_BOUNDS)


def _scprep_body(gidx_hbm, dst_hbm, cpk_hbm, cnt_hbm,
                 gst_v, dst_v, cpk_v, cw_v, sg0, sg1, sd0, sd1):
    c = lax.axis_index("c")
    s = lax.axis_index("s")
    w = c * 16 + s
    lo = w * RPT
    lanes = lax.iota(jnp.int32, 16)
    lane15 = jnp.full((16,), 15, jnp.int32)
    sg = [sg0, sg1]
    sd = [sd0, sd1]

    NB = EP // SB

    def start(blk, b):
        pltpu.async_copy(gidx_hbm.at[pl.ds(blk * SB, SB)],
                         gst_v.at[b], sg[b])
        pltpu.async_copy(dst_hbm.at[pl.ds(blk * SB, SB)],
                         dst_v.at[b], sd[b])

    def wait(blk, b):
        pltpu.make_async_copy(gidx_hbm.at[pl.ds(blk * SB, SB)],
                              gst_v.at[b], sg[b]).wait()
        pltpu.make_async_copy(dst_hbm.at[pl.ds(blk * SB, SB)],
                              dst_v.at[b], sd[b]).wait()

    start(0, 0)

    def pair(g, off):
        for b in range(2):
            blk = g * 2 + b
            wait(blk, b)

            @pl.when(blk + 1 < NB)
            def _():
                start(blk + 1, 1 - b)

            def grp(k, off):
                for u in range(4):
                    kk = k * 4 + u
                    d16 = dst_v[b, pl.ds(kk * 16, 16)]
                    g16 = gst_v[b, pl.ds(kk * 16, 16)]
                    dl = d16 - lo
                    m = (dl >= 0) & (dl < RPT)
                    pk = g16 * 512 + dl
                    x = jnp.where(m, 1, 0).astype(jnp.int32)
                    for sh in (1, 2, 4, 8):
                        gv = _lane_gather(x, jnp.maximum(lanes - sh, 0))
                        x = x + jnp.where(lanes >= sh, gv, 0)
                    plsc.store_scatter(cpk_v, [off + x - 1], pk, mask=m)
                    off = off + _lane_gather(x, lane15)
                return off

            off = lax.fori_loop(0, SB // 64, grp, off)
        return off

    cnt = lax.fori_loop(0, NB // 2, pair, jnp.zeros((16,), jnp.int32))

    dummy = jnp.full((16,), RPT, jnp.int32)
    ones = jnp.full((16,), True)
    for t in range(CH // 16):
        plsc.store_scatter(cpk_v, [cnt + t * 16 + lanes], dummy, mask=ones)

    pltpu.sync_copy(cpk_v, cpk_hbm.at[w])
    cw_v[...] = cnt
    pltpu.sync_copy(cw_v, cnt_hbm.at[w])


def _sc_prep(gidx, dstp):
    fn = pl.kernel(
        _scprep_body,
        mesh=_mesh(),
        compiler_params=pltpu.CompilerParams(needs_layout_passes=False),
        out_type=[jax.ShapeDtypeStruct((NW, CAP), jnp.int32),
                  jax.ShapeDtypeStruct((NW, 16), jnp.int32)],
        scratch_types=[
            pltpu.VMEM((2, SB), jnp.int32),
            pltpu.VMEM((2, SB), jnp.int32),
            pltpu.VMEM((CAP,), jnp.int32),
            pltpu.VMEM((16,), jnp.int32),
            pltpu.SemaphoreType.DMA,
            pltpu.SemaphoreType.DMA,
            pltpu.SemaphoreType.DMA,
            pltpu.SemaphoreType.DMA,
        ],
    )
    return fn(gidx, dstp)


def _scstep_body(cpk_hbm, cnt_hbm, table_hbm, out_hbm,
                 pk_v, gl_v, rows_v, cw_v, acc_v, s0, s1):
    c = lax.axis_index("c")
    s = lax.axis_index("s")
    w = c * 16 + s
    sems = [s0, s1]

    z16 = jnp.zeros((16,), _f32)

    def zr(i, carry):
        for u in range(16):
            acc_v[pl.ds(i * 256 + u * 16, 16)] = z16
        return carry

    lax.fori_loop(0, (RPT + 1) * DOUT // 256, zr, 0)

    pltpu.sync_copy(cnt_hbm.at[w], cw_v)
    cnt = cw_v[...][0]
    nch = (cnt + CH - 1) // CH

    pltpu.sync_copy(cpk_hbm.at[w], pk_v)

    def unp(i, carry):
        gl_v[pl.ds(i * 16, 16)] = pk_v[pl.ds(i * 16, 16)] >> 9
        return carry

    lax.fori_loop(0, CAP // 16, unp, 0)

    def start(ci, b):
        pltpu.async_copy(table_hbm.at[gl_v.at[pl.ds(ci * CH, CH)]],
                         rows_v.at[b], sems[b])

    def wait(ci, b):
        pltpu.make_async_copy(table_hbm.at[gl_v.at[pl.ds(ci * CH, CH)]],
                              rows_v.at[b], sems[b]).wait()

    @pl.when(nch > 0)
    def _():
        start(0, 0)

    def pair(g, carry):
        for b in range(2):
            ci = g * 2 + b

            @pl.when(ci < nch)
            def _():
                wait(ci, b)

                @pl.when(ci + 1 < nch)
                def _():
                    start(ci + 1, 1 - b)

                def grp(gi, carry):
                    pk16 = pk_v[pl.ds(ci * CH + gi * 16, 16)]
                    d16 = (pk16 & 511) * DOUT
                    for l in range(16):
                        doff = d16[l]
                        xs = [rows_v[b, gi * 16 + l, pl.ds(j * 16, 16)]
                              for j in range(DOUT // 16)]
                        for j in range(DOUT // 16):
                            plsc.addupdate(
                                acc_v.at[pl.ds(doff + j * 16, 16)], xs[j])
                    return carry

                lax.fori_loop(0, CH // 16, grp, 0)
        return carry

    lax.fori_loop(0, (nch + 1) // 2, pair, 0)

    pltpu.sync_copy(acc_v.at[pl.ds(0, RPT * DOUT)],
                    out_hbm.at[pl.ds(w * RPT * DOUT, RPT * DOUT)])


def _sc_step(cpk, cnts, table_flat):
    fn = pl.kernel(
        _scstep_body,
        mesh=_mesh(),
        compiler_params=pltpu.CompilerParams(needs_layout_passes=False),
        out_type=jax.ShapeDtypeStruct((NOUT * DOUT,), _f32),
        scratch_types=[
            pltpu.VMEM((CAP,), jnp.int32),
            pltpu.VMEM((CAP,), jnp.int32),
            pltpu.VMEM((2, CH, DOUT), _f32),
            pltpu.VMEM((16,), jnp.int32),
            pltpu.VMEM(((RPT + 1) * DOUT,), _f32),
            pltpu.SemaphoreType.DMA,
            pltpu.SemaphoreType.DMA,
        ],
    )
    return fn(cpk, cnts, table_flat)


def _message_pass(cpk, cnts, table_flat):
    return _sc_step(cpk, cnts, table_flat).reshape(NOUT, DOUT)[:N]


L1 = LG - 2
L1P = (L1 - 3) // 2 + 1
L2P = (L1P - 2) // 2 + 1
CNT1 = NG * L1
CNT2 = NG * L1P
EPS = 1e-5


def _conv3(x, w_ref, b_ref, width):
    acc = b_ref[...] * jnp.ones((L1, width), _f32)
    for d in range(3):
        acc = acc + _dot(x[d:d + L1], w_ref[d])
    return acc


def _stats_acc(g, y, sum_ref, sq_ref):
    @pl.when(g == 0)
    def _():
        sum_ref[...] = jnp.zeros_like(sum_ref)
        sq_ref[...] = jnp.zeros_like(sq_ref)
    sum_ref[...] += jnp.sum(y, axis=0, keepdims=True)
    sq_ref[...] += jnp.sum(y * y, axis=0, keepdims=True)


def _r1_body(h_ref, x_ref, w1_ref, b1_ref, wc1_ref, bc1_ref,
             y1_ref, z1_ref, ys_ref, yq_ref, zs_ref, zq_ref):
    g = pl.program_id(0)
    hb = h_ref[0]
    xb = x_ref[0]
    y = _conv3(hb, w1_ref, b1_ref, DOUT)
    cb = jnp.concatenate([hb, xb], axis=1)
    z = _conv3(cb, wc1_ref, bc1_ref, CC)
    y1_ref[0] = y
    z1_ref[0] = z
    _stats_acc(g, y, ys_ref, yq_ref)
    _stats_acc(g, z, zs_ref, zq_ref)


def _bn_relu(x, s_ref, q_ref, g_ref, b_ref, cnt):
    m = s_ref[...] / cnt
    v = q_ref[...] / cnt - m * m
    scale = g_ref[...] * lax.rsqrt(v + EPS)
    return jnp.maximum((x - m) * scale + b_ref[...], 0.0)


def _pool32(x, lout):
    m1 = jnp.max(x[:2 * lout].reshape(lout, 2, -1), axis=1)
    m2 = x[2:2 + 2 * lout].reshape(lout, 2, -1)[:, 0]
    return jnp.maximum(m1, m2)


def _pool22(x, lout):
    return jnp.max(x[:2 * lout].reshape(lout, 2, -1), axis=1)


def _r2_body(y1_ref, z1_ref, ys_ref, yq_ref, zs_ref, zq_ref,
             bng_ref, bnb_ref, bncg_ref, bncb_ref, w2_ref, wc2_ref,
             b2_ref, bc2_ref,
             y2_ref, z2_ref, ys2_ref, yq2_ref, zs2_ref, zq2_ref):
    g = pl.program_id(0)
    yn = _bn_relu(y1_ref[0], ys_ref, yq_ref, bng_ref, bnb_ref, float(CNT1))
    zn = _bn_relu(z1_ref[0], zs_ref, zq_ref, bncg_ref, bncb_ref, float(CNT1))
    yp = _pool32(yn, L1P)
    zp = _pool32(zn, L1P)
    y2 = _dot(yp, w2_ref[...]) + b2_ref[...]
    z2 = _dot(zp, wc2_ref[...]) + bc2_ref[...]
    y2_ref[0] = y2
    z2_ref[0] = z2
    _stats_acc(g, y2, ys2_ref, yq2_ref)
    _stats_acc(g, z2, zs2_ref, zq2_ref)


def _r3_body(y2_ref, z2_ref, ys2_ref, yq2_ref, zs2_ref, zq2_ref,
             bng_ref, bnb_ref, bncg_ref, bncb_ref,
             mly_ref, mlyb_ref, mlz_ref, mlzb_ref, out_ref):
    yn = _bn_relu(y2_ref[0], ys2_ref, yq2_ref, bng_ref, bnb_ref, float(CNT2))
    zn = _bn_relu(z2_ref[0], zs2_ref, zq2_ref, bncg_ref, bncb_ref, float(CNT2))
    yp = _pool22(yn, L2P)
    zp = _pool22(zn, L2P)
    yv = jnp.sum(yp * mly_ref[0][None, :], axis=1) + mlyb_ref[0, 0]
    zv = jnp.sum(zp * mlz_ref[0][None, :], axis=1) + mlzb_ref[0, 0]
    avg = jnp.mean(yv * zv)
    out_ref[...] = jnp.full((1, 8, 128), jax.nn.sigmoid(avg), _f32)


def _g_spec(l, cols):
    return pl.BlockSpec((1, l, cols), lambda g: (g, 0, 0))


def _readout(h, features, conv1_w, conv1_b, conv2_w, conv2_b,
             convc1_w, convc1_b, convc2_w, convc2_b,
             bn_g, bn_b, bnc_g, bnc_b, mly_w, mly_b, mlz_w, mlz_b):
    h_i = h.reshape(NG, LG, DOUT)
    x_i = features.reshape(NG, LG, DIN)
    w1t = jnp.transpose(conv1_w, (2, 0, 1))
    wc1t = jnp.transpose(convc1_w, (2, 0, 1))
    w2 = conv2_w[:, :, 0]
    wc2 = convc2_w[:, :, 0]

    y1, z1, ys, yq, zs, zq = pl.pallas_call(
        _r1_body,
        grid=(NG,),
        in_specs=[_g_spec(LG, DOUT), _g_spec(LG, DIN),
                  _full((3, DOUT, DOUT)), _full((1, DOUT)),
                  _full((3, CC, CC)), _full((1, CC))],
        out_specs=[_g_spec(L1, DOUT), _g_spec(L1, CC),
                   _full((1, DOUT)), _full((1, DOUT)),
                   _full((1, CC)), _full((1, CC))],
        out_shape=[jax.ShapeDtypeStruct((NG, L1, DOUT), _f32),
                   jax.ShapeDtypeStruct((NG, L1, CC), _f32),
                   jax.ShapeDtypeStruct((1, DOUT), _f32),
                   jax.ShapeDtypeStruct((1, DOUT), _f32),
                   jax.ShapeDtypeStruct((1, CC), _f32),
                   jax.ShapeDtypeStruct((1, CC), _f32)],
    )(h_i, x_i, w1t, conv1_b.reshape(1, DOUT), wc1t, convc1_b.reshape(1, CC))

    y2, z2, ys2, yq2, zs2, zq2 = pl.pallas_call(
        _r2_body,
        grid=(NG,),
        in_specs=[_g_spec(L1, DOUT), _g_spec(L1, CC),
                  _full((1, DOUT)), _full((1, DOUT)),
                  _full((1, CC)), _full((1, CC)),
                  _full((1, DOUT)), _full((1, DOUT)),
                  _full((1, CC)), _full((1, CC)),
                  _full((DOUT, DOUT)), _full((CC, CC)),
                  _full((1, DOUT)), _full((1, CC))],
        out_specs=[_g_spec(L1P, DOUT), _g_spec(L1P, CC),
                   _full((1, DOUT)), _full((1, DOUT)),
                   _full((1, CC)), _full((1, CC))],
        out_shape=[jax.ShapeDtypeStruct((NG, L1P, DOUT), _f32),
                   jax.ShapeDtypeStruct((NG, L1P, CC), _f32),
                   jax.ShapeDtypeStruct((1, DOUT), _f32),
                   jax.ShapeDtypeStruct((1, DOUT), _f32),
                   jax.ShapeDtypeStruct((1, CC), _f32),
                   jax.ShapeDtypeStruct((1, CC), _f32)],
    )(y1, z1, ys, yq, zs, zq,
      bn_g.reshape(1, DOUT), bn_b.reshape(1, DOUT),
      bnc_g.reshape(1, CC), bnc_b.reshape(1, CC),
      w2, wc2, conv2_b.reshape(1, DOUT), convc2_b.reshape(1, CC))

    out = pl.pallas_call(
        _r3_body,
        grid=(NG,),
        in_specs=[_g_spec(L1P, DOUT), _g_spec(L1P, CC),
                  _full((1, DOUT)), _full((1, DOUT)),
                  _full((1, CC)), _full((1, CC)),
                  _full((1, DOUT)), _full((1, DOUT)),
                  _full((1, CC)), _full((1, CC)),
                  _full((1, DOUT)), _full((1, 128)),
                  _full((1, CC)), _full((1, 128))],
        out_specs=pl.BlockSpec((1, 8, 128), lambda g: (g, 0, 0)),
        out_shape=jax.ShapeDtypeStruct((NG, 8, 128), _f32),
    )(y2, z2, ys2, yq2, zs2, zq2,
      bn_g.reshape(1, DOUT), bn_b.reshape(1, DOUT),
      bnc_g.reshape(1, CC), bnc_b.reshape(1, CC),
      mly_w, jnp.broadcast_to(mly_b.reshape(1, 1), (1, 128)),
      mlz_w, jnp.broadcast_to(mlz_b.reshape(1, 1), (1, 128)))

    return out[:, 0, 0]


def kernel(features, edge_index, edge_types, num_graphs, W_e, b_e, W_ih,
           W_hh, b_ih, b_hh, conv1_w, conv1_b, conv2_w, conv2_b, convc1_w,
           convc1_b, convc2_w, convc2_b, bn_g, bn_b, bnc_g, bnc_b, mly_w,
           mly_b, mlz_w, mlz_b):
    src = jnp.pad(edge_index[0].astype(jnp.int32), (0, EP - E))
    dstp = jnp.pad(edge_index[1].astype(jnp.int32), (0, EP - E),
                   constant_values=PADV)
    et = jnp.pad(edge_types.astype(jnp.int32), (0, EP - E))

    gidx = _prep(src, et).reshape(EP)
    cpk, cnts = _sc_prep(gidx, dstp)

    b_ih2 = b_ih.reshape(1, 3 * DOUT)
    b_hh2 = b_hh.reshape(1, 3 * DOUT)

    h, table = _init_call(features, W_e, b_e)
    for step in range(STEPS):
        a = _message_pass(cpk, cnts, table.reshape(NET * N, DOUT))
        if step < STEPS - 1:
            h, table = _gru_mnode_call(a, h, W_ih, W_hh, b_ih2, b_hh2,
                                       W_e, b_e)
        else:
            h = _gru_call(a, h, W_ih, W_hh, b_ih2, b_hh2)

    return _readout(h, features, conv1_w, conv1_b, conv2_w, conv2_b,
                    convc1_w, convc1_b, convc2_w, convc2_b,
                    bn_g, bn_b, bnc_g, bnc_b, mly_w, mly_b, mlz_w, mlz_b)
